# baseline (speedup 1.0000x reference)
import functools

import jax
import jax.numpy as jnp
from jax import lax
from jax.experimental import pallas as pl
from jax.experimental.pallas import tpu as pltpu
from jax.experimental.pallas import tpu_sc as plsc

N_NODES = 10000
NC = 2
NS = 16
NW = NC * NS
HALF = 5120
TRASH = HALF
ACC_ELEMS = (HALF + 8) * 16
DEG_ELEMS = HALF + 16
CHUNK = 128


def _sc_segment_sum(edge_attr, dst, zeros):
    E = edge_attr.shape[0]
    per_w = E // NW
    n_chunks = per_w // CHUNK
    tail = per_w - n_chunks * CHUNK
    assert per_w * NW == E and tail % 16 == 0

    mesh = plsc.VectorSubcoreMesh(core_axis_name="c", subcore_axis_name="s")

    @functools.partial(
        pl.kernel,
        out_type=(
            jax.ShapeDtypeStruct((2 * NW * HALF * 16,), jnp.float32),
            jax.ShapeDtypeStruct((2 * NW * HALF,), jnp.float32),
        ),
        mesh=mesh,
        compiler_params=pltpu.CompilerParams(needs_layout_passes=False),
        scratch_types=[
            pltpu.VMEM((CHUNK, 16), jnp.float32),
            pltpu.VMEM((CHUNK,), jnp.int32),
            pltpu.VMEM((ACC_ELEMS,), jnp.float32),
            pltpu.VMEM((DEG_ELEMS,), jnp.float32),
        ],
    )
    def sc_kernel(ea_hbm, dst_hbm, zeros_hbm, feat_out, deg_out,
                  ea_v, idx_v, acc_v, deg_v):
        c = lax.axis_index("c")
        s = lax.axis_index("s")
        wid = s * NC + c
        base = wid * per_w

        cols = lax.iota(jnp.int32, 16)
        lane0 = cols == 0
        ones16 = jnp.ones((16,), jnp.float32)

        def do_group(g0, lo):
            iv = idx_v[pl.ds(g0, 16)]
            for e in range(16):
                dv = jnp.broadcast_to(iv[e], (16,))
                rows = jnp.minimum((dv - lo).astype(jnp.uint32),
                                   jnp.uint32(TRASH)).astype(jnp.int32)
                plsc.addupdate_scatter(acc_v, [rows * 16 + cols],
                                       ea_v[g0 + e, :])
                plsc.addupdate_scatter(deg_v, [rows], ones16,
                                       mask=lane0 & (rows != TRASH))

        for p in range(2):
            lo = p * HALF
            pltpu.sync_copy(zeros_hbm, acc_v.at[pl.ds(0, HALF * 16)])
            pltpu.sync_copy(zeros_hbm.at[pl.ds(0, HALF)],
                            deg_v.at[pl.ds(0, HALF)])

            def chunk_body(j, carry):
                off = base + j * CHUNK
                pltpu.sync_copy(dst_hbm.at[pl.ds(off, CHUNK)], idx_v)
                pltpu.sync_copy(ea_hbm.at[pl.ds(off, CHUNK)], ea_v)
                for g in range(CHUNK // 16):
                    do_group(g * 16, lo)
                return carry

            lax.fori_loop(0, n_chunks, chunk_body, 0)

            if tail:
                off = base + n_chunks * CHUNK
                pltpu.sync_copy(dst_hbm.at[pl.ds(off, tail)],
                                idx_v.at[pl.ds(0, tail)])
                pltpu.sync_copy(ea_hbm.at[pl.ds(off, tail)],
                                ea_v.at[pl.ds(0, tail)])
                for g in range(tail // 16):
                    do_group(g * 16, lo)

            slot = p * NW + wid
            pltpu.sync_copy(acc_v.at[pl.ds(0, HALF * 16)],
                            feat_out.at[pl.ds(slot * HALF * 16,
                                              HALF * 16)])
            pltpu.sync_copy(deg_v.at[pl.ds(0, HALF)],
                            deg_out.at[pl.ds(slot * HALF, HALF)])

    return sc_kernel(edge_attr, dst, zeros)


def _tc_merge(feats, degs):
    def body(f_ref, d_ref, fo_ref, do_ref, facc, dacc):
        j = pl.program_id(0)

        @pl.when(j == 0)
        def _():
            facc[...] = jnp.zeros_like(facc)
            dacc[...] = jnp.zeros_like(dacc)

        p = j // NW
        facc[p] += f_ref[0]
        dacc[p] += d_ref[0, 0]

        @pl.when(j == 2 * NW - 1)
        def _():
            fo_ref[...] = facc[...]
            do_ref[...] = dacc[...]

    return pl.pallas_call(
        body,
        grid=(2 * NW,),
        in_specs=[pl.BlockSpec((1, HALF, 16), lambda j: (j, 0, 0)),
                  pl.BlockSpec((1, 1, HALF), lambda j: (j, 0, 0))],
        out_specs=[pl.BlockSpec((2, HALF, 16), lambda j: (0, 0, 0)),
                   pl.BlockSpec((2, HALF), lambda j: (0, 0))],
        out_shape=[jax.ShapeDtypeStruct((2, HALF, 16), jnp.float32),
                   jax.ShapeDtypeStruct((2, HALF), jnp.float32)],
        scratch_shapes=[pltpu.VMEM((2, HALF, 16), jnp.float32),
                        pltpu.VMEM((2, HALF), jnp.float32)],
    )(feats, degs)


def _tc_dense(x, merged, deg_col, W_atom, b_atom, W_bond, b_bond, gamma,
              beta):
    n = x.shape[0]

    def body(x_ref, m_ref, deg_ref, wa_ref, ba_ref, wb_ref, bb_ref, g_ref,
             be_ref, o_ref):
        agg = jnp.concatenate([m_ref[0], m_ref[1]], axis=0)[:n]
        h = (jnp.dot(agg, wb_ref[...], preferred_element_type=jnp.float32)
             + deg_ref[...] * bb_ref[...]
             + jnp.dot(x_ref[...], wa_ref[...],
                       preferred_element_type=jnp.float32)
             + ba_ref[...])
        mean = jnp.mean(h, axis=0, keepdims=True)
        d = h - mean
        var = jnp.mean(d * d, axis=0, keepdims=True)
        hn = d * lax.rsqrt(var + 1e-5)
        o_ref[...] = jnp.maximum(g_ref[...] * hn + be_ref[...], 0.0)

    return pl.pallas_call(
        body,
        out_shape=jax.ShapeDtypeStruct((n, W_atom.shape[1]), jnp.float32),
    )(x, merged, deg_col, W_atom, b_atom, W_bond, b_bond, gamma, beta)


def kernel(x, edge_index, edge_attr, W_atom, b_atom, W_bond, b_bond,
           gamma, beta):
    dst = edge_index[1].astype(jnp.int32)
    zeros = jnp.zeros((HALF * 16,), jnp.float32)

    feats, degs = _sc_segment_sum(edge_attr, dst, zeros)
    merged, deg_m = _tc_merge(feats.reshape(2 * NW, HALF, 16),
                              degs.reshape(2 * NW, 1, HALF))
    deg_col = deg_m.reshape(2 * HALF)[:N_NODES, None]

    return _tc_dense(x, merged, deg_col,
                     W_atom, b_atom.reshape(1, -1),
                     W_bond, b_bond.reshape(1, -1),
                     gamma.reshape(1, -1), beta.reshape(1, -1))


# speedup vs baseline: 1.1335x; 1.1335x over previous
import functools

import jax
import jax.numpy as jnp
from jax import lax
from jax.experimental import pallas as pl
from jax.experimental.pallas import tpu as pltpu
from jax.experimental.pallas import tpu_sc as plsc

N_NODES = 10000
NC = 2
NS = 16
NW = NC * NS
HALF = 5120
TRASH = HALF
ACC_ELEMS = (HALF + 8) * 16
DEG_ELEMS = HALF + 16
CHUNK = 256


def _sc_segment_sum(edge_attr, dst, zeros):
    E = edge_attr.shape[0]
    per_w = E // NW
    n_chunks = per_w // CHUNK
    tail = per_w - n_chunks * CHUNK
    assert per_w * NW == E and tail % 16 == 0

    mesh = plsc.VectorSubcoreMesh(core_axis_name="c", subcore_axis_name="s")

    @functools.partial(
        pl.kernel,
        out_type=(
            jax.ShapeDtypeStruct((2 * NW * HALF * 16,), jnp.float32),
            jax.ShapeDtypeStruct((2 * NW * HALF,), jnp.float32),
        ),
        mesh=mesh,
        compiler_params=pltpu.CompilerParams(needs_layout_passes=False),
        scratch_types=[
            pltpu.VMEM((CHUNK, 16), jnp.float32),
            pltpu.VMEM((CHUNK,), jnp.int32),
            pltpu.VMEM((ACC_ELEMS,), jnp.float32),
            pltpu.VMEM((DEG_ELEMS,), jnp.float32),
        ],
    )
    def sc_kernel(ea_hbm, dst_hbm, zeros_hbm, feat_out, deg_out,
                  ea_v, idx_v, acc_v, deg_v):
        c = lax.axis_index("c")
        s = lax.axis_index("s")
        wid = s * NC + c
        base = wid * per_w

        cols = lax.iota(jnp.int32, 16)
        lane0 = cols == 0
        ones16 = jnp.ones((16,), jnp.float32)

        def do_group(g0, lo):
            iv = idx_v[pl.ds(g0, 16)]
            for e in range(16):
                dv = jnp.broadcast_to(iv[e], (16,))
                rows = jnp.minimum((dv - lo).astype(jnp.uint32),
                                   jnp.uint32(TRASH)).astype(jnp.int32)
                plsc.addupdate_scatter(acc_v, [rows * 16 + cols],
                                       ea_v[g0 + e, :])
                plsc.addupdate_scatter(deg_v, [rows], ones16,
                                       mask=lane0 & (rows != TRASH))

        for p in range(2):
            lo = p * HALF
            pltpu.sync_copy(zeros_hbm, acc_v.at[pl.ds(0, HALF * 16)])
            pltpu.sync_copy(zeros_hbm.at[pl.ds(0, HALF)],
                            deg_v.at[pl.ds(0, HALF)])

            def chunk_body(j, carry):
                off = base + j * CHUNK
                pltpu.sync_copy(dst_hbm.at[pl.ds(off, CHUNK)], idx_v)
                pltpu.sync_copy(ea_hbm.at[pl.ds(off, CHUNK)], ea_v)
                for g in range(CHUNK // 16):
                    do_group(g * 16, lo)
                return carry

            lax.fori_loop(0, n_chunks, chunk_body, 0)

            if tail:
                off = base + n_chunks * CHUNK
                pltpu.sync_copy(dst_hbm.at[pl.ds(off, tail)],
                                idx_v.at[pl.ds(0, tail)])
                pltpu.sync_copy(ea_hbm.at[pl.ds(off, tail)],
                                ea_v.at[pl.ds(0, tail)])
                for g in range(tail // 16):
                    do_group(g * 16, lo)

            slot = p * NW + wid
            pltpu.sync_copy(acc_v.at[pl.ds(0, HALF * 16)],
                            feat_out.at[pl.ds(slot * HALF * 16,
                                              HALF * 16)])
            pltpu.sync_copy(deg_v.at[pl.ds(0, HALF)],
                            deg_out.at[pl.ds(slot * HALF, HALF)])

    return sc_kernel(edge_attr, dst, zeros)


def _tc_merge(feats, degs):
    def body(f_ref, d_ref, fo_ref, do_ref, facc, dacc):
        j = pl.program_id(0)

        @pl.when(j == 0)
        def _():
            facc[...] = jnp.zeros_like(facc)
            dacc[...] = jnp.zeros_like(dacc)

        p = j // NW
        facc[p] += f_ref[0]
        dacc[p] += d_ref[0, 0]

        @pl.when(j == 2 * NW - 1)
        def _():
            fo_ref[...] = facc[...]
            do_ref[...] = dacc[...]

    return pl.pallas_call(
        body,
        grid=(2 * NW,),
        in_specs=[pl.BlockSpec((1, HALF, 16), lambda j: (j, 0, 0)),
                  pl.BlockSpec((1, 1, HALF), lambda j: (j, 0, 0))],
        out_specs=[pl.BlockSpec((2, HALF, 16), lambda j: (0, 0, 0)),
                   pl.BlockSpec((2, HALF), lambda j: (0, 0))],
        out_shape=[jax.ShapeDtypeStruct((2, HALF, 16), jnp.float32),
                   jax.ShapeDtypeStruct((2, HALF), jnp.float32)],
        scratch_shapes=[pltpu.VMEM((2, HALF, 16), jnp.float32),
                        pltpu.VMEM((2, HALF), jnp.float32)],
    )(feats, degs)


def _tc_dense(x, merged, deg_col, W_atom, b_atom, W_bond, b_bond, gamma,
              beta):
    n = x.shape[0]

    def body(x_ref, m_ref, deg_ref, wa_ref, ba_ref, wb_ref, bb_ref, g_ref,
             be_ref, o_ref):
        agg = jnp.concatenate([m_ref[0], m_ref[1]], axis=0)[:n]
        h = (jnp.dot(agg, wb_ref[...], preferred_element_type=jnp.float32)
             + deg_ref[...] * bb_ref[...]
             + jnp.dot(x_ref[...], wa_ref[...],
                       preferred_element_type=jnp.float32)
             + ba_ref[...])
        mean = jnp.mean(h, axis=0, keepdims=True)
        d = h - mean
        var = jnp.mean(d * d, axis=0, keepdims=True)
        hn = d * lax.rsqrt(var + 1e-5)
        o_ref[...] = jnp.maximum(g_ref[...] * hn + be_ref[...], 0.0)

    return pl.pallas_call(
        body,
        out_shape=jax.ShapeDtypeStruct((n, W_atom.shape[1]), jnp.float32),
    )(x, merged, deg_col, W_atom, b_atom, W_bond, b_bond, gamma, beta)


def kernel(x, edge_index, edge_attr, W_atom, b_atom, W_bond, b_bond,
           gamma, beta):
    dst = edge_index[1].astype(jnp.int32)
    zeros = jnp.zeros((HALF * 16,), jnp.float32)

    feats, degs = _sc_segment_sum(edge_attr, dst, zeros)
    merged, deg_m = _tc_merge(feats.reshape(2 * NW, HALF, 16),
                              degs.reshape(2 * NW, 1, HALF))
    deg_col = deg_m.reshape(2 * HALF)[:N_NODES, None]

    return _tc_dense(x, merged, deg_col,
                     W_atom, b_atom.reshape(1, -1),
                     W_bond, b_bond.reshape(1, -1),
                     gamma.reshape(1, -1), beta.reshape(1, -1))


# speedup vs baseline: 1.1606x; 1.0239x over previous
import functools

import jax
import jax.numpy as jnp
from jax import lax
from jax.experimental import pallas as pl
from jax.experimental.pallas import tpu as pltpu
from jax.experimental.pallas import tpu_sc as plsc

N_NODES = 10000
NC = 2
NS = 16
NW = NC * NS
HALF = 5120
TRASH = HALF
ACC_ELEMS = (HALF + 8) * 16
DEG_ELEMS = HALF + 16
CHUNK = 320


def _sc_segment_sum(edge_attr, dst, zeros):
    E = edge_attr.shape[0]
    per_w = E // NW
    n_chunks = per_w // CHUNK
    tail = per_w - n_chunks * CHUNK
    assert per_w * NW == E and tail % 16 == 0

    mesh = plsc.VectorSubcoreMesh(core_axis_name="c", subcore_axis_name="s")

    @functools.partial(
        pl.kernel,
        out_type=(
            jax.ShapeDtypeStruct((2 * NW * HALF * 16,), jnp.float32),
            jax.ShapeDtypeStruct((2 * NW * HALF,), jnp.float32),
        ),
        mesh=mesh,
        compiler_params=pltpu.CompilerParams(needs_layout_passes=False),
        scratch_types=[
            pltpu.VMEM((CHUNK, 16), jnp.float32),
            pltpu.VMEM((CHUNK,), jnp.int32),
            pltpu.VMEM((ACC_ELEMS,), jnp.float32),
            pltpu.VMEM((DEG_ELEMS,), jnp.float32),
        ],
    )
    def sc_kernel(ea_hbm, dst_hbm, zeros_hbm, feat_out, deg_out,
                  ea_v, idx_v, acc_v, deg_v):
        c = lax.axis_index("c")
        s = lax.axis_index("s")
        wid = s * NC + c
        base = wid * per_w

        cols = lax.iota(jnp.int32, 16)
        lane0 = cols == 0
        ones16 = jnp.ones((16,), jnp.float32)

        def do_group(g0, lo):
            iv = idx_v[pl.ds(g0, 16)]
            for e in range(16):
                dv = jnp.broadcast_to(iv[e], (16,))
                rows = jnp.minimum((dv - lo).astype(jnp.uint32),
                                   jnp.uint32(TRASH)).astype(jnp.int32)
                plsc.addupdate_scatter(acc_v, [rows * 16 + cols],
                                       ea_v[g0 + e, :])
                plsc.addupdate_scatter(deg_v, [rows], ones16,
                                       mask=lane0 & (rows != TRASH))

        for p in range(2):
            lo = p * HALF
            pltpu.sync_copy(zeros_hbm, acc_v.at[pl.ds(0, HALF * 16)])
            pltpu.sync_copy(zeros_hbm.at[pl.ds(0, HALF)],
                            deg_v.at[pl.ds(0, HALF)])

            def chunk_body(j, carry):
                off = base + j * CHUNK
                pltpu.sync_copy(dst_hbm.at[pl.ds(off, CHUNK)], idx_v)
                pltpu.sync_copy(ea_hbm.at[pl.ds(off, CHUNK)], ea_v)
                for g in range(CHUNK // 16):
                    do_group(g * 16, lo)
                return carry

            lax.fori_loop(0, n_chunks, chunk_body, 0)

            if tail:
                off = base + n_chunks * CHUNK
                pltpu.sync_copy(dst_hbm.at[pl.ds(off, tail)],
                                idx_v.at[pl.ds(0, tail)])
                pltpu.sync_copy(ea_hbm.at[pl.ds(off, tail)],
                                ea_v.at[pl.ds(0, tail)])
                for g in range(tail // 16):
                    do_group(g * 16, lo)

            slot = p * NW + wid
            pltpu.sync_copy(acc_v.at[pl.ds(0, HALF * 16)],
                            feat_out.at[pl.ds(slot * HALF * 16,
                                              HALF * 16)])
            pltpu.sync_copy(deg_v.at[pl.ds(0, HALF)],
                            deg_out.at[pl.ds(slot * HALF, HALF)])

    return sc_kernel(edge_attr, dst, zeros)


def _tc_merge(feats, degs):
    def body(f_ref, d_ref, fo_ref, do_ref, facc, dacc):
        j = pl.program_id(0)

        @pl.when(j == 0)
        def _():
            facc[...] = jnp.zeros_like(facc)
            dacc[...] = jnp.zeros_like(dacc)

        p = j // NW
        facc[p] += f_ref[0]
        dacc[p] += d_ref[0, 0]

        @pl.when(j == 2 * NW - 1)
        def _():
            fo_ref[...] = facc[...]
            do_ref[...] = dacc[...]

    return pl.pallas_call(
        body,
        grid=(2 * NW,),
        in_specs=[pl.BlockSpec((1, HALF, 16), lambda j: (j, 0, 0)),
                  pl.BlockSpec((1, 1, HALF), lambda j: (j, 0, 0))],
        out_specs=[pl.BlockSpec((2, HALF, 16), lambda j: (0, 0, 0)),
                   pl.BlockSpec((2, HALF), lambda j: (0, 0))],
        out_shape=[jax.ShapeDtypeStruct((2, HALF, 16), jnp.float32),
                   jax.ShapeDtypeStruct((2, HALF), jnp.float32)],
        scratch_shapes=[pltpu.VMEM((2, HALF, 16), jnp.float32),
                        pltpu.VMEM((2, HALF), jnp.float32)],
    )(feats, degs)


def _tc_dense(x, merged, deg_col, W_atom, b_atom, W_bond, b_bond, gamma,
              beta):
    n = x.shape[0]

    def body(x_ref, m_ref, deg_ref, wa_ref, ba_ref, wb_ref, bb_ref, g_ref,
             be_ref, o_ref):
        agg = jnp.concatenate([m_ref[0], m_ref[1]], axis=0)[:n]
        h = (jnp.dot(agg, wb_ref[...], preferred_element_type=jnp.float32)
             + deg_ref[...] * bb_ref[...]
             + jnp.dot(x_ref[...], wa_ref[...],
                       preferred_element_type=jnp.float32)
             + ba_ref[...])
        mean = jnp.mean(h, axis=0, keepdims=True)
        d = h - mean
        var = jnp.mean(d * d, axis=0, keepdims=True)
        hn = d * lax.rsqrt(var + 1e-5)
        o_ref[...] = jnp.maximum(g_ref[...] * hn + be_ref[...], 0.0)

    return pl.pallas_call(
        body,
        out_shape=jax.ShapeDtypeStruct((n, W_atom.shape[1]), jnp.float32),
    )(x, merged, deg_col, W_atom, b_atom, W_bond, b_bond, gamma, beta)


def kernel(x, edge_index, edge_attr, W_atom, b_atom, W_bond, b_bond,
           gamma, beta):
    dst = edge_index[1].astype(jnp.int32)
    zeros = jnp.zeros((HALF * 16,), jnp.float32)

    feats, degs = _sc_segment_sum(edge_attr, dst, zeros)
    merged, deg_m = _tc_merge(feats.reshape(2 * NW, HALF, 16),
                              degs.reshape(2 * NW, 1, HALF))
    deg_col = deg_m.reshape(2 * HALF)[:N_NODES, None]

    return _tc_dense(x, merged, deg_col,
                     W_atom, b_atom.reshape(1, -1),
                     W_bond, b_bond.reshape(1, -1),
                     gamma.reshape(1, -1), beta.reshape(1, -1))
